# baseline (speedup 1.0000x reference)
import functools

import jax
import jax.numpy as jnp
from jax import lax
from jax.experimental import pallas as pl
from jax.experimental.pallas import tpu as pltpu
from jax.experimental.pallas import tpu_sc as plsc

Q = 16384
N = 4096
KNN = 11
POOL = 14
TQ = 512


def _cand_kernel(p_ref, ct_ref, idx_ref):
    d2 = jnp.dot(p_ref[...], ct_ref[...],
                 preferred_element_type=jnp.float32,
                 precision=jax.lax.Precision.HIGHEST)
    iota = lax.broadcasted_iota(jnp.int32, (TQ, N), 1)
    kb = lax.bitcast_convert_type(d2, jnp.int32)
    key = lax.bitcast_convert_type((kb & jnp.int32(-4096)) | iota,
                                   jnp.float32)
    cols = []
    for k in range(POOL):
        m = jnp.min(key, axis=1, keepdims=True)
        cols.append(m)
        if k + 1 < POOL:
            key = jnp.where(key <= m, jnp.float32(jnp.inf), key)
    cols += [cols[0]] * (16 - POOL)
    packed = jnp.concatenate(cols, axis=1)
    idx_ref[...] = lax.bitcast_convert_type(packed, jnp.int32) & jnp.int32(0xFFF)


def _lex_lt(da, ia, db, ib):
    return (da < db) | ((da == db) & (ia < ib))


def _sc_edge_body(px_h, py_h, pz_h, cx_h, cy_h, cz_h, idx_h, out_h,
                  px_v, py_v, pz_v, cx_v, cy_v, cz_v, idx_v, out_v):
    info = plsc.get_sparse_core_info()
    nc, ns, L = info.num_cores, info.num_subcores, info.num_lanes
    qpw = Q // (nc * ns)
    wid = lax.axis_index("s") * nc + lax.axis_index("c")
    base = wid * qpw

    pltpu.sync_copy(px_h.at[pl.ds(base, qpw)], px_v)
    pltpu.sync_copy(py_h.at[pl.ds(base, qpw)], py_v)
    pltpu.sync_copy(pz_h.at[pl.ds(base, qpw)], pz_v)
    pltpu.sync_copy(cx_h, cx_v)
    pltpu.sync_copy(cy_h, cy_v)
    pltpu.sync_copy(cz_h, cz_v)
    pltpu.sync_copy(idx_h.at[pl.ds(base * 16, qpw * 16)], idx_v)

    lane = lax.broadcasted_iota(jnp.int32, (L,), 0)
    inf = jnp.full((L,), jnp.inf, jnp.float32)
    ninf = jnp.full((L,), -jnp.inf, jnp.float32)
    false = lane < 0

    def step(j, _):
        qoff = j * L
        px = px_v[pl.ds(qoff, L)]
        py = py_v[pl.ds(qoff, L)]
        pz = pz_v[pl.ds(qoff, L)]
        ibase = qoff * 16 + lane * 16
        iks, d2s = [], []
        for k in range(POOL):
            ik = plsc.load_gather(idx_v, [ibase + k])
            dx = px - plsc.load_gather(cx_v, [ik])
            dy = py - plsc.load_gather(cy_v, [ik])
            dz = pz - plsc.load_gather(cz_v, [ik])
            iks.append(ik)
            d2s.append((dx * dx + dy * dy) + dz * dz)
        d0, i0 = d2s[0], iks[0]
        for k in range(1, POOL):
            lt = _lex_lt(d2s[k], iks[k], d0, i0)
            d0 = jnp.where(lt, d2s[k], d0)
            i0 = jnp.where(lt, iks[k], i0)
        excl = [false] * POOL
        for _ in range(POOL - KNN):
            dm, im = ninf, lane
            for k in range(POOL):
                dk = jnp.where(excl[k], ninf, d2s[k])
                gt = _lex_lt(dm, im, dk, iks[k])
                dm = jnp.where(gt, dk, dm)
                im = jnp.where(gt, iks[k], im)
            for k in range(POOL):
                excl[k] = excl[k] | ((d2s[k] == dm) & (iks[k] == im))
        c0x = plsc.load_gather(cx_v, [i0])
        c0y = plsc.load_gather(cy_v, [i0])
        c0z = plsc.load_gather(cz_v, [i0])
        pcx = px - c0x
        pcy = py - c0y
        pcz = pz - c0z
        best = inf
        for k in range(POOL):
            ik = iks[k]
            ex = plsc.load_gather(cx_v, [ik]) - c0x
            ey = plsc.load_gather(cy_v, [ik]) - c0y
            ez = plsc.load_gather(cz_v, [ik]) - c0z
            el2 = ex * ex + ey * ey + ez * ez
            dt = pcx * ex + pcy * ey + pcz * ez
            sq = dt * dt / el2 - dt + el2 * 0.25
            skip = excl[k] | ((d2s[k] == d0) & (ik == i0))
            best = jnp.minimum(best, jnp.where(skip, inf, sq))
        out_v[pl.ds(qoff, L)] = best
        return 0

    lax.fori_loop(0, qpw // L, step, 0)
    pltpu.sync_copy(out_v, out_h.at[pl.ds(base, qpw)])


def _sc_edge(px, py, pz, cx, cy, cz, idx_flat):
    info = plsc.get_sparse_core_info()
    qpw = Q // (info.num_cores * info.num_subcores)
    return pl.kernel(
        _sc_edge_body,
        out_type=jax.ShapeDtypeStruct((Q,), jnp.float32),
        mesh=plsc.VectorSubcoreMesh(core_axis_name="c", subcore_axis_name="s"),
        compiler_params=pltpu.CompilerParams(needs_layout_passes=False),
        scratch_types=[
            pltpu.VMEM((qpw,), jnp.float32),
            pltpu.VMEM((qpw,), jnp.float32),
            pltpu.VMEM((qpw,), jnp.float32),
            pltpu.VMEM((N,), jnp.float32),
            pltpu.VMEM((N,), jnp.float32),
            pltpu.VMEM((N,), jnp.float32),
            pltpu.VMEM((qpw * 16,), jnp.int32),
            pltpu.VMEM((qpw,), jnp.float32),
        ],
    )(px, py, pz, cx, cy, cz, idx_flat)


@jax.jit
def kernel(points, cell_points):
    pn = jnp.sum(points * points, axis=1)
    cn = jnp.sum(cell_points * cell_points, axis=1)
    p8 = (jnp.zeros((Q, 8), jnp.float32)
          .at[:, :3].set(-2.0 * points)
          .at[:, 3].set(pn)
          .at[:, 4].set(1.0))
    ct8 = (jnp.zeros((8, N), jnp.float32)
           .at[:3, :].set(cell_points.T)
           .at[3, :].set(1.0)
           .at[4, :].set(cn))
    idx = pl.pallas_call(
        _cand_kernel,
        grid=(Q // TQ,),
        in_specs=[
            pl.BlockSpec((TQ, 8), lambda i: (i, 0)),
            pl.BlockSpec((8, N), lambda i: (0, 0)),
        ],
        out_specs=pl.BlockSpec((TQ, 16), lambda i: (i, 0)),
        out_shape=jax.ShapeDtypeStruct((Q, 16), jnp.int32),
    )(p8, ct8)
    px, py, pz = points[:, 0], points[:, 1], points[:, 2]
    cx, cy, cz = cell_points[:, 0], cell_points[:, 1], cell_points[:, 2]
    return _sc_edge(px, py, pz, cx, cy, cz, idx.reshape(-1))


# speedup vs baseline: 1.1009x; 1.1009x over previous
import functools

import jax
import jax.numpy as jnp
from jax import lax
from jax.experimental import pallas as pl
from jax.experimental.pallas import tpu as pltpu
from jax.experimental.pallas import tpu_sc as plsc

Q = 16384
N = 4096
KNN = 11
POOL = 14
TQ = 512


def _cand_kernel(p_ref, ct_ref, idx_ref):
    p = p_ref[...]
    ct = ct_ref[...]
    cn = jnp.sum(ct * ct, axis=0, keepdims=True)
    pn = jnp.sum(p * p, axis=1, keepdims=True)
    d2 = (pn + cn) - 2.0 * jnp.dot(p, ct, preferred_element_type=jnp.float32,
                                   precision=jax.lax.Precision.HIGHEST)
    iota = lax.broadcasted_iota(jnp.int32, (TQ, N), 1)
    kb = lax.bitcast_convert_type(d2, jnp.int32)
    key = lax.bitcast_convert_type((kb & jnp.int32(-4096)) | iota,
                                   jnp.float32)
    cols = []
    for k in range(POOL):
        m = jnp.min(key, axis=1, keepdims=True)
        cols.append(m)
        if k + 1 < POOL:
            key = jnp.where(key <= m, jnp.float32(jnp.inf), key)
    cols += [cols[0]] * (16 - POOL)
    packed = jnp.concatenate(cols, axis=1)
    idx_ref[...] = lax.bitcast_convert_type(packed, jnp.int32) & jnp.int32(0xFFF)


def _lex_lt(da, ia, db, ib):
    return (da < db) | ((da == db) & (ia < ib))


def _sc_edge_body(px_h, py_h, pz_h, cx_h, cy_h, cz_h, idx_h, out_h,
                  px_v, py_v, pz_v, cx_v, cy_v, cz_v, idx_v, out_v):
    info = plsc.get_sparse_core_info()
    nc, ns, L = info.num_cores, info.num_subcores, info.num_lanes
    qpw = Q // (nc * ns)
    wid = lax.axis_index("s") * nc + lax.axis_index("c")
    base = wid * qpw

    pltpu.sync_copy(px_h.at[pl.ds(base, qpw)], px_v)
    pltpu.sync_copy(py_h.at[pl.ds(base, qpw)], py_v)
    pltpu.sync_copy(pz_h.at[pl.ds(base, qpw)], pz_v)
    pltpu.sync_copy(cx_h, cx_v)
    pltpu.sync_copy(cy_h, cy_v)
    pltpu.sync_copy(cz_h, cz_v)
    pltpu.sync_copy(idx_h.at[pl.ds(base * 16, qpw * 16)], idx_v)

    lane = lax.broadcasted_iota(jnp.int32, (L,), 0)
    inf = jnp.full((L,), jnp.inf, jnp.float32)
    ninf = jnp.full((L,), -jnp.inf, jnp.float32)
    false = lane < 0

    def step(j, _):
        qoff = j * L
        px = px_v[pl.ds(qoff, L)]
        py = py_v[pl.ds(qoff, L)]
        pz = pz_v[pl.ds(qoff, L)]
        ibase = qoff * 16 + lane * 16
        iks, d2s = [], []
        for k in range(POOL):
            ik = plsc.load_gather(idx_v, [ibase + k])
            dx = px - plsc.load_gather(cx_v, [ik])
            dy = py - plsc.load_gather(cy_v, [ik])
            dz = pz - plsc.load_gather(cz_v, [ik])
            iks.append(ik)
            d2s.append((dx * dx + dy * dy) + dz * dz)
        d0, i0 = d2s[0], iks[0]
        for k in range(1, POOL):
            lt = _lex_lt(d2s[k], iks[k], d0, i0)
            d0 = jnp.where(lt, d2s[k], d0)
            i0 = jnp.where(lt, iks[k], i0)
        excl = [false] * POOL
        for _ in range(POOL - KNN):
            dm, im = ninf, lane
            for k in range(POOL):
                dk = jnp.where(excl[k], ninf, d2s[k])
                gt = _lex_lt(dm, im, dk, iks[k])
                dm = jnp.where(gt, dk, dm)
                im = jnp.where(gt, iks[k], im)
            for k in range(POOL):
                excl[k] = excl[k] | ((d2s[k] == dm) & (iks[k] == im))
        c0x = plsc.load_gather(cx_v, [i0])
        c0y = plsc.load_gather(cy_v, [i0])
        c0z = plsc.load_gather(cz_v, [i0])
        pcx = px - c0x
        pcy = py - c0y
        pcz = pz - c0z
        best = inf
        for k in range(POOL):
            ik = iks[k]
            ex = plsc.load_gather(cx_v, [ik]) - c0x
            ey = plsc.load_gather(cy_v, [ik]) - c0y
            ez = plsc.load_gather(cz_v, [ik]) - c0z
            el2 = ex * ex + ey * ey + ez * ez
            dt = pcx * ex + pcy * ey + pcz * ez
            sq = dt * dt / el2 - dt + el2 * 0.25
            skip = excl[k] | ((d2s[k] == d0) & (ik == i0))
            best = jnp.minimum(best, jnp.where(skip, inf, sq))
        out_v[pl.ds(qoff, L)] = best
        return 0

    lax.fori_loop(0, qpw // L, step, 0)
    pltpu.sync_copy(out_v, out_h.at[pl.ds(base, qpw)])


def _sc_edge(px, py, pz, cx, cy, cz, idx_flat):
    info = plsc.get_sparse_core_info()
    qpw = Q // (info.num_cores * info.num_subcores)
    return pl.kernel(
        _sc_edge_body,
        out_type=jax.ShapeDtypeStruct((Q,), jnp.float32),
        mesh=plsc.VectorSubcoreMesh(core_axis_name="c", subcore_axis_name="s"),
        compiler_params=pltpu.CompilerParams(needs_layout_passes=False),
        scratch_types=[
            pltpu.VMEM((qpw,), jnp.float32),
            pltpu.VMEM((qpw,), jnp.float32),
            pltpu.VMEM((qpw,), jnp.float32),
            pltpu.VMEM((N,), jnp.float32),
            pltpu.VMEM((N,), jnp.float32),
            pltpu.VMEM((N,), jnp.float32),
            pltpu.VMEM((qpw * 16,), jnp.int32),
            pltpu.VMEM((qpw,), jnp.float32),
        ],
    )(px, py, pz, cx, cy, cz, idx_flat)


@jax.jit
def kernel(points, cell_points):
    p8 = jnp.zeros((Q, 8), jnp.float32).at[:, :3].set(points)
    ct8 = jnp.zeros((8, N), jnp.float32).at[:3, :].set(cell_points.T)
    idx = pl.pallas_call(
        _cand_kernel,
        grid=(Q // TQ,),
        in_specs=[
            pl.BlockSpec((TQ, 8), lambda i: (i, 0)),
            pl.BlockSpec((8, N), lambda i: (0, 0)),
        ],
        out_specs=pl.BlockSpec((TQ, 16), lambda i: (i, 0)),
        out_shape=jax.ShapeDtypeStruct((Q, 16), jnp.int32),
    )(p8, ct8)
    px, py, pz = points[:, 0], points[:, 1], points[:, 2]
    cx, cy, cz = cell_points[:, 0], cell_points[:, 1], cell_points[:, 2]
    return _sc_edge(px, py, pz, cx, cy, cz, idx.reshape(-1))


# speedup vs baseline: 1.1207x; 1.0180x over previous
import functools

import jax
import jax.numpy as jnp
from jax import lax
from jax.experimental import pallas as pl
from jax.experimental.pallas import tpu as pltpu
from jax.experimental.pallas import tpu_sc as plsc

Q = 16384
N = 4096
KNN = 11
POOL = 14
TQ = 512


def _cand_kernel(p_ref, ct_ref, idx_ref):
    p = p_ref[...]
    ct = ct_ref[...]
    cn = jnp.sum(ct * ct, axis=0, keepdims=True)
    pn = jnp.sum(p * p, axis=1, keepdims=True)
    d2 = (pn + cn) - 2.0 * jnp.dot(p, ct, preferred_element_type=jnp.float32,
                                   precision=jax.lax.Precision.HIGHEST)
    iota = lax.broadcasted_iota(jnp.int32, (TQ, N), 1)
    kb = lax.bitcast_convert_type(d2, jnp.int32)
    key = lax.bitcast_convert_type((kb & jnp.int32(-4096)) | iota,
                                   jnp.float32)
    cols = []
    for k in range(POOL):
        m = jnp.min(key, axis=1, keepdims=True)
        cols.append(m)
        if k + 1 < POOL:
            key = jnp.where(key <= m, jnp.float32(jnp.inf), key)
    cols += [cols[0]] * (16 - POOL)
    packed = jnp.concatenate(cols, axis=1)
    idx_ref[...] = lax.bitcast_convert_type(packed, jnp.int32) & jnp.int32(0xFFF)


def _lex_lt(da, ia, db, ib):
    return (da < db) | ((da == db) & (ia < ib))


def _sc_edge_body(px_h, py_h, pz_h, cx_h, cy_h, cz_h, idx_h, out_h,
                  px_v, py_v, pz_v, cx_v, cy_v, cz_v, idx_v, out_v):
    info = plsc.get_sparse_core_info()
    nc, ns, L = info.num_cores, info.num_subcores, info.num_lanes
    qpw = Q // (nc * ns)
    wid = lax.axis_index("s") * nc + lax.axis_index("c")
    base = wid * qpw

    pltpu.sync_copy(px_h.at[pl.ds(base, qpw)], px_v)
    pltpu.sync_copy(py_h.at[pl.ds(base, qpw)], py_v)
    pltpu.sync_copy(pz_h.at[pl.ds(base, qpw)], pz_v)
    pltpu.sync_copy(cx_h, cx_v)
    pltpu.sync_copy(cy_h, cy_v)
    pltpu.sync_copy(cz_h, cz_v)
    pltpu.sync_copy(idx_h.at[pl.ds(base * 16, qpw * 16)], idx_v)

    lane = lax.broadcasted_iota(jnp.int32, (L,), 0)
    inf = jnp.full((L,), jnp.inf, jnp.float32)
    ninf = jnp.full((L,), -jnp.inf, jnp.float32)
    false = lane < 0

    def step(j, _):
        qoff = j * L
        px = px_v[pl.ds(qoff, L)]
        py = py_v[pl.ds(qoff, L)]
        pz = pz_v[pl.ds(qoff, L)]
        ibase = qoff * 16 + lane * 16
        iks, d2s = [], []
        for k in range(POOL):
            ik = plsc.load_gather(idx_v, [ibase + k])
            dx = px - plsc.load_gather(cx_v, [ik])
            dy = py - plsc.load_gather(cy_v, [ik])
            dz = pz - plsc.load_gather(cz_v, [ik])
            iks.append(ik)
            d2s.append((dx * dx + dy * dy) + dz * dz)
        d0, i0 = d2s[0], iks[0]
        for k in range(1, POOL):
            lt = _lex_lt(d2s[k], iks[k], d0, i0)
            d0 = jnp.where(lt, d2s[k], d0)
            i0 = jnp.where(lt, iks[k], i0)
        excl = [false] * POOL
        for _ in range(POOL - KNN):
            dm, im = ninf, lane
            for k in range(POOL):
                dk = jnp.where(excl[k], ninf, d2s[k])
                gt = _lex_lt(dm, im, dk, iks[k])
                dm = jnp.where(gt, dk, dm)
                im = jnp.where(gt, iks[k], im)
            for k in range(POOL):
                excl[k] = excl[k] | ((d2s[k] == dm) & (iks[k] == im))
        c0x = plsc.load_gather(cx_v, [i0])
        c0y = plsc.load_gather(cy_v, [i0])
        c0z = plsc.load_gather(cz_v, [i0])
        pcx = px - c0x
        pcy = py - c0y
        pcz = pz - c0z
        best = inf
        for k in range(POOL):
            ik = iks[k]
            ex = plsc.load_gather(cx_v, [ik]) - c0x
            ey = plsc.load_gather(cy_v, [ik]) - c0y
            ez = plsc.load_gather(cz_v, [ik]) - c0z
            el2 = ex * ex + ey * ey + ez * ez
            dt = pcx * ex + pcy * ey + pcz * ez
            sq = dt * dt / el2 - dt + el2 * 0.25
            skip = excl[k] | ((d2s[k] == d0) & (ik == i0))
            best = jnp.minimum(best, jnp.where(skip, inf, sq))
        out_v[pl.ds(qoff, L)] = best
        return 0

    lax.fori_loop(0, qpw // L, step, 0)
    pltpu.sync_copy(out_v, out_h.at[pl.ds(base, qpw)])


def _sc_edge(px, py, pz, cx, cy, cz, idx_flat):
    info = plsc.get_sparse_core_info()
    qpw = Q // (info.num_cores * info.num_subcores)
    return pl.kernel(
        _sc_edge_body,
        out_type=jax.ShapeDtypeStruct((Q,), jnp.float32),
        mesh=plsc.VectorSubcoreMesh(core_axis_name="c", subcore_axis_name="s"),
        compiler_params=pltpu.CompilerParams(needs_layout_passes=False),
        scratch_types=[
            pltpu.VMEM((qpw,), jnp.float32),
            pltpu.VMEM((qpw,), jnp.float32),
            pltpu.VMEM((qpw,), jnp.float32),
            pltpu.VMEM((N,), jnp.float32),
            pltpu.VMEM((N,), jnp.float32),
            pltpu.VMEM((N,), jnp.float32),
            pltpu.VMEM((qpw * 16,), jnp.int32),
            pltpu.VMEM((qpw,), jnp.float32),
        ],
    )(px, py, pz, cx, cy, cz, idx_flat)


@jax.jit
def kernel(points, cell_points):
    ct3 = cell_points.T
    idx = pl.pallas_call(
        _cand_kernel,
        grid=(Q // TQ,),
        in_specs=[
            pl.BlockSpec((TQ, 3), lambda i: (i, 0)),
            pl.BlockSpec((3, N), lambda i: (0, 0)),
        ],
        out_specs=pl.BlockSpec((TQ, 16), lambda i: (i, 0)),
        out_shape=jax.ShapeDtypeStruct((Q, 16), jnp.int32),
    )(points, ct3)
    px, py, pz = points[:, 0], points[:, 1], points[:, 2]
    cx, cy, cz = cell_points[:, 0], cell_points[:, 1], cell_points[:, 2]
    return _sc_edge(px, py, pz, cx, cy, cz, idx.reshape(-1))
